# baseline (speedup 1.0000x reference)
import jax
import jax.numpy as jnp
from jax import lax
from jax.experimental import pallas as pl
from jax.experimental.pallas import tpu as pltpu
from jax.experimental.pallas import tpu_sc as plsc

F = 16
B = 4
HO = 512
WO = 512
N = B * HO * WO
QB = HO * WO
NW = 32
NPW = N // NW
C = 512
NCHUNK = NPW // C
NG = C // 16
LEVEL_SIZES = (1024, 512, 256, 128)


def _tex_kernel(gx_hbm, gy_hbm, t1, t2, t3, t4, out_hbm, *scr):
    (gxb, gyb,
     accs,
     bufs,
     wr,
     idxr,
     semg0, semg1, semc, semo0, semo1) = scr
    tables = (t1, t2, t3, t4)
    semg = (semg0, semg1)
    semo = (semo0, semo1)
    cid = lax.axis_index("c")
    sid = lax.axis_index("s")
    wid = sid * 2 + cid

    def compute_idx(li, gx_ref, gy_ref):
        s = LEVEL_SIZES[li]
        sf = float(s)

        def body(g, _):
            g16 = g * 16
            gxv = gx_ref[pl.ds(g16, 16)]
            gyv = gy_ref[pl.ds(g16, 16)]
            ix = jnp.clip(gxv * (sf * 0.5) + (sf - 1.0) * 0.5, 0.0, sf - 1.0)
            iy = jnp.clip(gyv * (sf * 0.5) + (sf - 1.0) * 0.5, 0.0, sf - 1.0)
            x0 = ix.astype(jnp.int32)
            y0 = iy.astype(jnp.int32)
            wx = ix - x0.astype(jnp.float32)
            wy = iy - y0.astype(jnp.float32)
            w11 = wx * wy
            w10 = wy - w11
            w01 = wx - w11
            wr[0][li, pl.ds(g16, 16)] = (1.0 - wx) - w10
            wr[1][li, pl.ds(g16, 16)] = w01
            wr[2][li, pl.ds(g16, 16)] = w10
            wr[3][li, pl.ds(g16, 16)] = w11
            x1 = jnp.minimum(x0 + 1, s - 1)
            y1 = jnp.minimum(y0 + 1, s - 1)
            r0 = y0 * s
            r1 = y1 * s
            idxr[li][0][pl.ds(g16, 16)] = r0 + x0
            idxr[li][1][pl.ds(g16, 16)] = r0 + x1
            idxr[li][2][pl.ds(g16, 16)] = r1 + x0
            idxr[li][3][pl.ds(g16, 16)] = r1 + x1
            return _

        lax.fori_loop(0, NG, body, None)

    def issue_gathers(li):
        st = li % 2
        for cn in range(4):
            pltpu.async_copy(tables[li].at[idxr[li][cn]], bufs[st][cn], semg[st])

    def drain_gathers(li):
        st = li % 2
        for cn in range(4):
            pltpu.make_async_copy(
                tables[li].at[idxr[li][cn]], bufs[st][cn], semg[st]).wait()

    def comp_level(li, acc):
        st = li % 2
        b00, b01, b10, b11 = bufs[st]

        def body(g, _):
            g16 = g * 16
            wv00 = wr[0][li, pl.ds(g16, 16)]
            wv01 = wr[1][li, pl.ds(g16, 16)]
            wv10 = wr[2][li, pl.ds(g16, 16)]
            wv11 = wr[3][li, pl.ds(g16, 16)]
            for j in range(16):
                p = g16 + j
                jv = jnp.full((16,), j, jnp.int32)
                w00 = jnp.take_along_axis(wv00, jv, axis=0, mode="promise_in_bounds")
                w01 = jnp.take_along_axis(wv01, jv, axis=0, mode="promise_in_bounds")
                w10 = jnp.take_along_axis(wv10, jv, axis=0, mode="promise_in_bounds")
                w11 = jnp.take_along_axis(wv11, jv, axis=0, mode="promise_in_bounds")
                contrib = ((w00 * b00[p, :] + w01 * b01[p, :])
                           + (w10 * b10[p, :] + w11 * b11[p, :]))
                if li == 0:
                    acc[p, :] = contrib
                else:
                    plsc.addupdate(acc.at[p, :], contrib)
            return _

        lax.fori_loop(0, NG, body, None)

    def chunk_body(c, par):
        base = wid * NPW + c * C
        nbase = jnp.minimum(base + C, N - C)
        npar = 1 - par
        cpx = pltpu.async_copy(gx_hbm.at[pl.ds(nbase, C)], gxb.at[npar], semc)
        cpy = pltpu.async_copy(gy_hbm.at[pl.ds(nbase, C)], gyb.at[npar], semc)
        acc = accs[par]
        for li in range(4):
            if li < 3:
                issue_gathers(li + 1)
            drain_gathers(li)
            comp_level(li, acc)
        cpx.wait()
        cpy.wait()
        for li in range(4):
            compute_idx(li, gxb.at[npar], gyb.at[npar])
        issue_gathers(0)
        @pl.when(c >= 2)
        def _():
            pltpu.make_async_copy(
                acc, out_hbm.at[pl.ds(0, C)], semo[par]).wait()
        pltpu.async_copy(acc, out_hbm.at[pl.ds(base, C)], semo[par])

    base0 = wid * NPW
    pltpu.sync_copy(gx_hbm.at[pl.ds(base0, C)], gxb.at[0])
    pltpu.sync_copy(gy_hbm.at[pl.ds(base0, C)], gyb.at[0])
    for li in range(4):
        compute_idx(li, gxb.at[0], gyb.at[0])
    issue_gathers(0)

    def pair_body(p, _):
        chunk_body(2 * p, 0)
        chunk_body(2 * p + 1, 1)
        return _

    lax.fori_loop(0, NCHUNK // 2, pair_body, None)

    drain_gathers(0)
    for par in range(2):
        pltpu.make_async_copy(
            accs[par], out_hbm.at[pl.ds(0, C)], semo[par]).wait()


@jax.jit
def kernel(x, L1, L2, L3, L4):
    gx = x[..., 0].reshape(N)
    gy = x[..., 1].reshape(N)
    tables = [jnp.transpose(t, (1, 2, 0)).reshape(-1, F)
              for t in (L1, L2, L3, L4)]

    mesh = plsc.VectorSubcoreMesh(core_axis_name="c", subcore_axis_name="s",
                                  num_cores=2, num_subcores=16)
    fn = pl.kernel(
        _tex_kernel,
        out_type=jax.ShapeDtypeStruct((N, F), jnp.float32),
        mesh=mesh,
        scratch_types=[
            pltpu.VMEM((2, C), jnp.float32),
            pltpu.VMEM((2, C), jnp.float32),
            [pltpu.VMEM((C, F), jnp.float32) for _ in range(2)],
            [[pltpu.VMEM((C, F), jnp.float32) for _ in range(4)]
             for _ in range(2)],
            [pltpu.VMEM((4, C), jnp.float32) for _ in range(4)],
            [[pltpu.VMEM((C,), jnp.int32) for _ in range(4)]
             for _ in range(4)],
            pltpu.SemaphoreType.DMA,
            pltpu.SemaphoreType.DMA,
            pltpu.SemaphoreType.DMA,
            pltpu.SemaphoreType.DMA,
            pltpu.SemaphoreType.DMA,
        ],
        compiler_params=pltpu.CompilerParams(needs_layout_passes=False,
                                             use_tc_tiling_on_sc=False,
                                             disable_bounds_checks=True),
    )
    out = fn(gx, gy, *tables)
    return jnp.transpose(out.reshape(B, QB, F), (0, 2, 1)).reshape(B, F, HO, WO)
